# initial kernel scaffold (unmeasured)
import jax
import jax.numpy as jnp
from jax import lax
from jax.experimental import pallas as pl
from jax.experimental.pallas import tpu as pltpu

N_DEV = 8
SQ_PER = 256
SQ = N_DEV * SQ_PER
SKV = 4096
HQ_PER = 8
DH = 128
DM = 1024
HD_PER = HQ_PER * DH
SCALE = 0.08838834764831843

_DeviceIdType = getattr(pl, "DeviceIdType", None) or pltpu.DeviceIdType
_sem_signal = getattr(pl, "semaphore_signal", None) or pltpu.semaphore_signal
_sem_wait = getattr(pl, "semaphore_wait", None) or pltpu.semaphore_wait
_CompilerParams = getattr(pltpu, "CompilerParams", None) or pltpu.TPUCompilerParams


def _neighbor_barrier(left, right):
    barrier_sem = pltpu.get_barrier_semaphore()
    for nbr in (left, right):
        _sem_signal(
            barrier_sem, inc=1,
            device_id=(nbr,), device_id_type=_DeviceIdType.MESH,
        )
    _sem_wait(barrier_sem, 2)


def _ring_allgather(x2):

    def body(x_ref, out_ref, comm_ref, send_sems, recv_sems):
        p = lax.axis_index("i")
        left = lax.rem(p - 1 + N_DEV, N_DEV)
        right = lax.rem(p + 1, N_DEV)
        _neighbor_barrier(left, right)

        out_ref[pl.ds(p * SQ_PER, SQ_PER), :] = x_ref[...]
        comm_ref[0] = x_ref[...]

        for h in range(N_DEV - 1):
            rdma = pltpu.make_async_remote_copy(
                src_ref=comm_ref.at[h],
                dst_ref=comm_ref.at[h + 1],
                send_sem=send_sems.at[h],
                recv_sem=recv_sems.at[h],
                device_id=(right,),
                device_id_type=_DeviceIdType.MESH,
            )
            rdma.start()
            rdma.wait()
            origin = lax.rem(p - h - 1 + N_DEV, N_DEV)
            out_ref[pl.ds(origin * SQ_PER, SQ_PER), :] = comm_ref[h + 1]

    return pl.pallas_call(
        body,
        out_shape=jax.ShapeDtypeStruct((SQ, DM), jnp.bfloat16),
        in_specs=[pl.BlockSpec(memory_space=pltpu.VMEM)],
        out_specs=pl.BlockSpec(memory_space=pltpu.VMEM),
        scratch_shapes=[
            pltpu.VMEM((N_DEV, SQ_PER, DM), jnp.bfloat16),
            pltpu.SemaphoreType.DMA((N_DEV - 1,)),
            pltpu.SemaphoreType.DMA((N_DEV - 1,)),
        ],
        compiler_params=_CompilerParams(collective_id=0),
    )(x2)


def _ring_reduce_scatter(partial):

    def body(p_ref, out_ref, sbuf, rbuf, send_sems, recv_sems):
        p = lax.axis_index("i")
        left = lax.rem(p - 1 + N_DEV, N_DEV)
        right = lax.rem(p + 1, N_DEV)
        _neighbor_barrier(left, right)

        c0 = lax.rem(p - 1 + N_DEV, N_DEV)
        sbuf[0] = p_ref[pl.ds(c0 * SQ_PER, SQ_PER), :]

        for s in range(N_DEV - 1):
            rdma = pltpu.make_async_remote_copy(
                src_ref=sbuf.at[s],
                dst_ref=rbuf.at[s],
                send_sem=send_sems.at[s],
                recv_sem=recv_sems.at[s],
                device_id=(right,),
                device_id_type=_DeviceIdType.MESH,
            )
            rdma.start()
            rdma.wait()
            rc = lax.rem(p - s - 2 + 2 * N_DEV, N_DEV)
            if s < N_DEV - 2:
                sbuf[s + 1] = rbuf[s] + p_ref[pl.ds(rc * SQ_PER, SQ_PER), :]
            else:
                out_ref[...] = rbuf[s] + p_ref[pl.ds(rc * SQ_PER, SQ_PER), :]

    return pl.pallas_call(
        body,
        out_shape=jax.ShapeDtypeStruct((SQ_PER, DM), jnp.float32),
        in_specs=[pl.BlockSpec(memory_space=pltpu.VMEM)],
        out_specs=pl.BlockSpec(memory_space=pltpu.VMEM),
        scratch_shapes=[
            pltpu.VMEM((N_DEV - 1, SQ_PER, DM), jnp.float32),
            pltpu.VMEM((N_DEV - 1, SQ_PER, DM), jnp.float32),
            pltpu.SemaphoreType.DMA((N_DEV - 1,)),
            pltpu.SemaphoreType.DMA((N_DEV - 1,)),
        ],
        compiler_params=_CompilerParams(collective_id=1),
    )(partial)


def kernel(x, Wq, K_ext, V_ext, Wo):
    p = lax.axis_index("i")

    x2 = x[0].astype(jnp.bfloat16)
    Wq_b = Wq.astype(jnp.bfloat16)
    Wo_b = Wo.astype(jnp.bfloat16)
    K_h = lax.dynamic_slice_in_dim(K_ext[0], p * HQ_PER, HQ_PER, axis=1)
    V_h = lax.dynamic_slice_in_dim(V_ext[0], p * HQ_PER, HQ_PER, axis=1)
    K_h = K_h.astype(jnp.bfloat16)
    V_h = V_h.astype(jnp.bfloat16)

    x_full = _ring_allgather(x2)

    Q = jnp.dot(x_full, Wq_b, preferred_element_type=jnp.float32)
    Q = Q.reshape(SQ, HQ_PER, DH).astype(jnp.bfloat16)

    scores = jnp.einsum(
        "qhd,khd->hqk", Q, K_h, preferred_element_type=jnp.float32
    ) * SCALE

    qi = jnp.arange(SQ)[:, None]
    ki = jnp.arange(SKV)[None, :]
    mask = (jnp.abs(qi - ki) <= 128) | (ki < 32) | (qi < 32)
    scores = jnp.where(mask[None], scores, -1e9)
    w = jax.nn.softmax(scores, axis=-1)

    ctx = jnp.einsum(
        "hqk,khd->qhd", w.astype(jnp.bfloat16), V_h,
        preferred_element_type=jnp.float32,
    ).reshape(SQ, HD_PER).astype(jnp.bfloat16)

    partial = jnp.dot(ctx, Wo_b, preferred_element_type=jnp.float32)

    out = _ring_reduce_scatter(partial)
    return out[None]


# baseline (device time: 266652 ns/iter reference)
import jax
import jax.numpy as jnp
from jax import lax
from jax.experimental import pallas as pl
from jax.experimental.pallas import tpu as pltpu

N_DEV = 8
SQ_PER = 256
SQ = N_DEV * SQ_PER
SKV = 4096
HQ_PER = 8
DH = 128
DM = 1024
HD_PER = HQ_PER * DH
WIN = 768
SCALE = 0.08838834764831843

_DeviceIdType = getattr(pl, "DeviceIdType", None) or pltpu.DeviceIdType
_sem_signal = getattr(pl, "semaphore_signal", None) or pltpu.semaphore_signal
_sem_wait = getattr(pl, "semaphore_wait", None) or pltpu.semaphore_wait
_CompilerParams = getattr(pltpu, "CompilerParams", None) or pltpu.TPUCompilerParams


def _neighbor_barrier(left, right):
    barrier_sem = pltpu.get_barrier_semaphore()
    for nbr in (left, right):
        _sem_signal(
            barrier_sem, inc=1,
            device_id=(nbr,), device_id_type=_DeviceIdType.MESH,
        )
    _sem_wait(barrier_sem, 2)


def _dot(a, b, dims):
    return jax.lax.dot_general(a, b, (dims, ((), ())),
                               preferred_element_type=jnp.float32)


def _ag_qproj(x2, Wq_b):

    def body(x_ref, wq_ref, q_ref, comm_ref, send_sems, recv_sems):
        p = lax.axis_index("i")
        left = lax.rem(p - 1 + N_DEV, N_DEV)
        right = lax.rem(p + 1, N_DEV)
        _neighbor_barrier(left, right)

        wq = wq_ref[...]
        comm_ref[0] = x_ref[...]
        for h in range(N_DEV - 1):
            rdma = pltpu.make_async_remote_copy(
                src_ref=comm_ref.at[h],
                dst_ref=comm_ref.at[h + 1],
                send_sem=send_sems.at[h],
                recv_sem=recv_sems.at[h],
                device_id=(right,),
                device_id_type=_DeviceIdType.MESH,
            )
            rdma.start()
            origin = lax.rem(p - h + N_DEV, N_DEV)
            q_ref[pl.ds(origin * SQ_PER, SQ_PER), :] = _dot(
                comm_ref[h], wq, ((1,), (0,))).astype(jnp.bfloat16)
            rdma.wait()
        origin = lax.rem(p + 1, N_DEV)
        q_ref[pl.ds(origin * SQ_PER, SQ_PER), :] = _dot(
            comm_ref[N_DEV - 1], wq, ((1,), (0,))).astype(jnp.bfloat16)

    return pl.pallas_call(
        body,
        out_shape=jax.ShapeDtypeStruct((SQ, HD_PER), jnp.bfloat16),
        in_specs=[pl.BlockSpec(memory_space=pltpu.VMEM)] * 2,
        out_specs=pl.BlockSpec(memory_space=pltpu.VMEM),
        scratch_shapes=[
            pltpu.VMEM((N_DEV, SQ_PER, DM), jnp.bfloat16),
            pltpu.SemaphoreType.DMA((N_DEV - 1,)),
            pltpu.SemaphoreType.DMA((N_DEV - 1,)),
        ],
        compiler_params=_CompilerParams(collective_id=0),
    )(x2, Wq_b)


def _attention(Q, K2, V2):

    def body(q_ref, k_ref, v_ref, o_ref):
        g = pl.program_id(1)
        qv = q_ref[...]
        qi = g * SQ_PER + jax.lax.broadcasted_iota(
            jnp.int32, (SQ_PER, 1), 0)

        @pl.when(g == 0)
        def _():
            s = _dot(qv, k_ref[...], ((1,), (1,))) * SCALE
            ki = jax.lax.broadcasted_iota(jnp.int32, (SQ_PER, SKV), 1)
            mask = (jnp.abs(qi - ki) <= 128) | (ki < 32) | (qi < 32)
            s = jnp.where(mask, s, -1e9)
            m = jnp.max(s, axis=1, keepdims=True)
            e = jnp.exp(s - m)
            w = (e / jnp.sum(e, axis=1, keepdims=True)).astype(jnp.bfloat16)
            o_ref[...] = _dot(w, v_ref[...], ((1,), (0,))).astype(
                jnp.bfloat16)

        @pl.when(g > 0)
        def _():
            w0 = (g - 1) * SQ_PER
            kw = k_ref[pl.ds(w0, WIN), :]
            k0 = k_ref[0:SQ_PER, :]
            s0 = _dot(qv, k0, ((1,), (1,))) * SCALE
            s1 = _dot(qv, kw, ((1,), (1,))) * SCALE
            ki0 = jax.lax.broadcasted_iota(jnp.int32, (SQ_PER, SQ_PER), 1)
            ki1 = w0 + jax.lax.broadcasted_iota(jnp.int32, (SQ_PER, WIN), 1)
            m0 = ((jnp.abs(qi - ki0) <= 128) | (ki0 < 32)) & (g != 1)
            m1 = (jnp.abs(qi - ki1) <= 128) | (ki1 < 32)
            s0 = jnp.where(m0, s0, -1e9)
            s1 = jnp.where(m1, s1, -1e9)
            s = jnp.concatenate([s0, s1], axis=1)
            m = jnp.max(s, axis=1, keepdims=True)
            e = jnp.exp(s - m)
            w = (e / jnp.sum(e, axis=1, keepdims=True)).astype(jnp.bfloat16)
            ctx = (_dot(w[:, :SQ_PER], v_ref[0:SQ_PER, :], ((1,), (0,)))
                   + _dot(w[:, SQ_PER:], v_ref[pl.ds(w0, WIN), :],
                          ((1,), (0,))))
            o_ref[...] = ctx.astype(jnp.bfloat16)

    return pl.pallas_call(
        body,
        grid=(HQ_PER, N_DEV),
        out_shape=jax.ShapeDtypeStruct((SQ, HD_PER), jnp.bfloat16),
        in_specs=[
            pl.BlockSpec((SQ_PER, DH), lambda h, g: (g, h)),
            pl.BlockSpec((SKV, DH), lambda h, g: (0, h)),
            pl.BlockSpec((SKV, DH), lambda h, g: (0, h)),
        ],
        out_specs=pl.BlockSpec((SQ_PER, DH), lambda h, g: (g, h)),
    )(Q, K2, V2)


def _rs_oproj(ctx, Wo_b):

    def body(c_ref, wo_ref, out_ref, sbuf, rbuf, send_sems, recv_sems):
        p = lax.axis_index("i")
        left = lax.rem(p - 1 + N_DEV, N_DEV)
        right = lax.rem(p + 1, N_DEV)
        _neighbor_barrier(left, right)

        wo = wo_ref[...]
        c0 = lax.rem(p - 1 + N_DEV, N_DEV)
        sbuf[0] = _dot(c_ref[pl.ds(c0 * SQ_PER, SQ_PER), :], wo, ((1,), (0,)))

        for s in range(N_DEV - 1):
            rdma = pltpu.make_async_remote_copy(
                src_ref=sbuf.at[s],
                dst_ref=rbuf.at[s],
                send_sem=send_sems.at[s],
                recv_sem=recv_sems.at[s],
                device_id=(right,),
                device_id_type=_DeviceIdType.MESH,
            )
            rdma.start()
            rc = lax.rem(p - s - 2 + 2 * N_DEV, N_DEV)
            own = _dot(c_ref[pl.ds(rc * SQ_PER, SQ_PER), :], wo, ((1,), (0,)))
            rdma.wait()
            if s < N_DEV - 2:
                sbuf[s + 1] = rbuf[s] + own
            else:
                out_ref[...] = rbuf[s] + own

    return pl.pallas_call(
        body,
        out_shape=jax.ShapeDtypeStruct((SQ_PER, DM), jnp.float32),
        in_specs=[pl.BlockSpec(memory_space=pltpu.VMEM)] * 2,
        out_specs=pl.BlockSpec(memory_space=pltpu.VMEM),
        scratch_shapes=[
            pltpu.VMEM((N_DEV - 1, SQ_PER, DM), jnp.float32),
            pltpu.VMEM((N_DEV - 1, SQ_PER, DM), jnp.float32),
            pltpu.SemaphoreType.DMA((N_DEV - 1,)),
            pltpu.SemaphoreType.DMA((N_DEV - 1,)),
        ],
        compiler_params=_CompilerParams(collective_id=1),
    )(ctx, Wo_b)


def kernel(x, Wq, K_ext, V_ext, Wo):
    p = lax.axis_index("i")

    x2 = x[0].astype(jnp.bfloat16)
    Wq_b = Wq.astype(jnp.bfloat16)
    Wo_b = Wo.astype(jnp.bfloat16)
    K_h = lax.dynamic_slice_in_dim(K_ext[0], p * HQ_PER, HQ_PER, axis=1)
    V_h = lax.dynamic_slice_in_dim(V_ext[0], p * HQ_PER, HQ_PER, axis=1)
    K2 = K_h.astype(jnp.bfloat16).reshape(SKV, HD_PER)
    V2 = V_h.astype(jnp.bfloat16).reshape(SKV, HD_PER)

    Q = _ag_qproj(x2, Wq_b)
    ctx = _attention(Q, K2, V2)
    out = _rs_oproj(ctx, Wo_b)
    return out[None]


# device time: 227941 ns/iter; 1.1698x vs baseline; 1.1698x over previous
import jax
import jax.numpy as jnp
from jax import lax
from jax.experimental import pallas as pl
from jax.experimental.pallas import tpu as pltpu

N_DEV = 8
SQ_PER = 256
SQ = N_DEV * SQ_PER
SKV = 4096
HQ_PER = 8
DH = 128
DM = 1024
HD_PER = HQ_PER * DH
WIN = 768
SCALE = 0.08838834764831843

_DeviceIdType = getattr(pl, "DeviceIdType", None) or pltpu.DeviceIdType
_sem_signal = getattr(pl, "semaphore_signal", None) or pltpu.semaphore_signal
_sem_wait = getattr(pl, "semaphore_wait", None) or pltpu.semaphore_wait
_CompilerParams = getattr(pltpu, "CompilerParams", None) or pltpu.TPUCompilerParams


def _neighbor_barrier(left, right):
    barrier_sem = pltpu.get_barrier_semaphore()
    for nbr in (left, right):
        _sem_signal(
            barrier_sem, inc=1,
            device_id=(nbr,), device_id_type=_DeviceIdType.MESH,
        )
    _sem_wait(barrier_sem, 2)


def _dot(a, b, dims):
    return jax.lax.dot_general(a, b, (dims, ((), ())),
                               preferred_element_type=jnp.float32)


def _ag_qproj(x2, Wq_b):

    def body(x_ref, wq_ref, q_ref, comm_ref, send_sems, recv_sems):
        p = lax.axis_index("i")
        left = lax.rem(p - 1 + N_DEV, N_DEV)
        right = lax.rem(p + 1, N_DEV)
        _neighbor_barrier(left, right)

        wq = wq_ref[...]
        comm_ref[0] = x_ref[...]
        for h in range(N_DEV - 1):
            rdma = pltpu.make_async_remote_copy(
                src_ref=comm_ref.at[h],
                dst_ref=comm_ref.at[h + 1],
                send_sem=send_sems.at[h],
                recv_sem=recv_sems.at[h],
                device_id=(right,),
                device_id_type=_DeviceIdType.MESH,
            )
            rdma.start()
            origin = lax.rem(p - h + N_DEV, N_DEV)
            q_ref[pl.ds(origin * SQ_PER, SQ_PER), :] = _dot(
                comm_ref[h], wq, ((1,), (0,))).astype(jnp.bfloat16)
            rdma.wait()
        origin = lax.rem(p + 1, N_DEV)
        q_ref[pl.ds(origin * SQ_PER, SQ_PER), :] = _dot(
            comm_ref[N_DEV - 1], wq, ((1,), (0,))).astype(jnp.bfloat16)

    return pl.pallas_call(
        body,
        out_shape=jax.ShapeDtypeStruct((SQ, HD_PER), jnp.bfloat16),
        in_specs=[pl.BlockSpec(memory_space=pltpu.VMEM)] * 2,
        out_specs=pl.BlockSpec(memory_space=pltpu.VMEM),
        scratch_shapes=[
            pltpu.VMEM((N_DEV, SQ_PER, DM), jnp.bfloat16),
            pltpu.SemaphoreType.DMA((N_DEV - 1,)),
            pltpu.SemaphoreType.DMA((N_DEV - 1,)),
        ],
        compiler_params=_CompilerParams(collective_id=0),
    )(x2, Wq_b)


def _attention(Q, K2, V2):

    def body(q_ref, k_ref, v_ref, o_ref):
        g = pl.program_id(1)
        qv = q_ref[...]
        qi = g * SQ_PER + jax.lax.broadcasted_iota(
            jnp.int32, (SQ_PER, 1), 0)

        @pl.when(g == 0)
        def _():
            s = _dot(qv, k_ref[...], ((1,), (1,))) * SCALE
            ki = jax.lax.broadcasted_iota(jnp.int32, (SQ_PER, SKV), 1)
            mask = (jnp.abs(qi - ki) <= 128) | (ki < 32) | (qi < 32)
            s = jnp.where(mask, s, -1e9)
            m = jnp.max(s, axis=1, keepdims=True)
            e = jnp.exp(s - m)
            w = (e / jnp.sum(e, axis=1, keepdims=True)).astype(jnp.bfloat16)
            o_ref[...] = _dot(w, v_ref[...], ((1,), (0,))).astype(
                jnp.bfloat16)

        @pl.when(g > 0)
        def _():
            w0 = (g - 1) * SQ_PER
            kw = k_ref[pl.ds(w0, WIN), :]
            k0 = k_ref[0:SQ_PER, :]
            s0 = _dot(qv, k0, ((1,), (1,))) * SCALE
            s1 = _dot(qv, kw, ((1,), (1,))) * SCALE
            ki0 = jax.lax.broadcasted_iota(jnp.int32, (SQ_PER, SQ_PER), 1)
            ki1 = w0 + jax.lax.broadcasted_iota(jnp.int32, (SQ_PER, WIN), 1)
            m0 = ((jnp.abs(qi - ki0) <= 128) | (ki0 < 32)) & (g != 1)
            m1 = (jnp.abs(qi - ki1) <= 128) | (ki1 < 32)
            s0 = jnp.where(m0, s0, -1e9)
            s1 = jnp.where(m1, s1, -1e9)
            s = jnp.concatenate([s0, s1], axis=1)
            m = jnp.max(s, axis=1, keepdims=True)
            e = jnp.exp(s - m)
            w = (e / jnp.sum(e, axis=1, keepdims=True)).astype(jnp.bfloat16)
            ctx = (_dot(w[:, :SQ_PER], v_ref[0:SQ_PER, :], ((1,), (0,)))
                   + _dot(w[:, SQ_PER:], v_ref[pl.ds(w0, WIN), :],
                          ((1,), (0,))))
            o_ref[...] = ctx.astype(jnp.bfloat16)

    return pl.pallas_call(
        body,
        grid=(HQ_PER, N_DEV),
        out_shape=jax.ShapeDtypeStruct((SQ, HD_PER), jnp.bfloat16),
        in_specs=[
            pl.BlockSpec((SQ_PER, DH), lambda h, g: (g, h)),
            pl.BlockSpec((SKV, DH), lambda h, g: (0, h)),
            pl.BlockSpec((SKV, DH), lambda h, g: (0, h)),
        ],
        out_specs=pl.BlockSpec((SQ_PER, DH), lambda h, g: (g, h)),
    )(Q, K2, V2)


def _rs_oproj(ctx, Wo_b):

    def body(c_ref, wo_ref, out_ref, sbuf, rbuf, send_sems, recv_sems):
        p = lax.axis_index("i")
        left = lax.rem(p - 1 + N_DEV, N_DEV)
        right = lax.rem(p + 1, N_DEV)
        _neighbor_barrier(left, right)

        wo = wo_ref[...]
        c0 = lax.rem(p - 1 + N_DEV, N_DEV)
        sbuf[0] = _dot(
            c_ref[pl.ds(c0 * SQ_PER, SQ_PER), :], wo, ((1,), (0,))
        ).astype(jnp.bfloat16)

        for s in range(N_DEV - 1):
            rdma = pltpu.make_async_remote_copy(
                src_ref=sbuf.at[s],
                dst_ref=rbuf.at[s],
                send_sem=send_sems.at[s],
                recv_sem=recv_sems.at[s],
                device_id=(right,),
                device_id_type=_DeviceIdType.MESH,
            )
            rdma.start()
            rc = lax.rem(p - s - 2 + 2 * N_DEV, N_DEV)
            own = _dot(c_ref[pl.ds(rc * SQ_PER, SQ_PER), :], wo, ((1,), (0,)))
            rdma.wait()
            if s < N_DEV - 2:
                sbuf[s + 1] = (rbuf[s].astype(jnp.float32) + own).astype(
                    jnp.bfloat16)
            else:
                out_ref[...] = rbuf[s].astype(jnp.float32) + own

    return pl.pallas_call(
        body,
        out_shape=jax.ShapeDtypeStruct((SQ_PER, DM), jnp.float32),
        in_specs=[pl.BlockSpec(memory_space=pltpu.VMEM)] * 2,
        out_specs=pl.BlockSpec(memory_space=pltpu.VMEM),
        scratch_shapes=[
            pltpu.VMEM((N_DEV - 1, SQ_PER, DM), jnp.bfloat16),
            pltpu.VMEM((N_DEV - 1, SQ_PER, DM), jnp.bfloat16),
            pltpu.SemaphoreType.DMA((N_DEV - 1,)),
            pltpu.SemaphoreType.DMA((N_DEV - 1,)),
        ],
        compiler_params=_CompilerParams(collective_id=1),
    )(ctx, Wo_b)


def kernel(x, Wq, K_ext, V_ext, Wo):
    p = lax.axis_index("i")

    x2 = x[0].astype(jnp.bfloat16)
    Wq_b = Wq.astype(jnp.bfloat16)
    Wo_b = Wo.astype(jnp.bfloat16)
    K_h = lax.dynamic_slice_in_dim(K_ext[0], p * HQ_PER, HQ_PER, axis=1)
    V_h = lax.dynamic_slice_in_dim(V_ext[0], p * HQ_PER, HQ_PER, axis=1)
    K2 = K_h.astype(jnp.bfloat16).reshape(SKV, HD_PER)
    V2 = V_h.astype(jnp.bfloat16).reshape(SKV, HD_PER)

    Q = _ag_qproj(x2, Wq_b)
    ctx = _attention(Q, K2, V2)
    out = _rs_oproj(ctx, Wo_b)
    return out[None]


# device time: 222013 ns/iter; 1.2011x vs baseline; 1.0267x over previous
import jax
import jax.numpy as jnp
from jax import lax
from jax.experimental import pallas as pl
from jax.experimental.pallas import tpu as pltpu

N_DEV = 8
SQ_PER = 256
SQ = N_DEV * SQ_PER
SKV = 4096
HQ_PER = 8
DH = 128
DM = 1024
HD_PER = HQ_PER * DH
WIN = 768
SCALE = 0.08838834764831843

_DeviceIdType = getattr(pl, "DeviceIdType", None) or pltpu.DeviceIdType
_sem_signal = getattr(pl, "semaphore_signal", None) or pltpu.semaphore_signal
_sem_wait = getattr(pl, "semaphore_wait", None) or pltpu.semaphore_wait
_CompilerParams = getattr(pltpu, "CompilerParams", None) or pltpu.TPUCompilerParams


def _neighbor_barrier(left, right):
    barrier_sem = pltpu.get_barrier_semaphore()
    for nbr in (left, right):
        _sem_signal(
            barrier_sem, inc=1,
            device_id=(nbr,), device_id_type=_DeviceIdType.MESH,
        )
    _sem_wait(barrier_sem, 2)


def _dot(a, b, dims):
    return jax.lax.dot_general(a, b, (dims, ((), ())),
                               preferred_element_type=jnp.float32)


def _ag_qproj(x2, Wq_b):

    def body(x_ref, wq_ref, q_ref, comm_ref, send_sems, recv_sems):
        p = lax.axis_index("i")
        left = lax.rem(p - 1 + N_DEV, N_DEV)
        right = lax.rem(p + 1, N_DEV)
        _neighbor_barrier(left, right)

        wq = wq_ref[...]
        comm_ref[0] = x_ref[...]
        for h in range(N_DEV - 1):
            rdma = pltpu.make_async_remote_copy(
                src_ref=comm_ref.at[h],
                dst_ref=comm_ref.at[h + 1],
                send_sem=send_sems.at[h],
                recv_sem=recv_sems.at[h],
                device_id=(right,),
                device_id_type=_DeviceIdType.MESH,
            )
            rdma.start()
            origin = lax.rem(p - h + N_DEV, N_DEV)
            q_ref[pl.ds(origin * SQ_PER, SQ_PER), :] = _dot(
                comm_ref[h], wq, ((1,), (0,))).astype(jnp.bfloat16)
            rdma.wait()
        origin = lax.rem(p + 1, N_DEV)
        q_ref[pl.ds(origin * SQ_PER, SQ_PER), :] = _dot(
            comm_ref[N_DEV - 1], wq, ((1,), (0,))).astype(jnp.bfloat16)

    return pl.pallas_call(
        body,
        out_shape=jax.ShapeDtypeStruct((SQ, HD_PER), jnp.bfloat16),
        in_specs=[pl.BlockSpec(memory_space=pltpu.VMEM)] * 2,
        out_specs=pl.BlockSpec(memory_space=pltpu.VMEM),
        scratch_shapes=[
            pltpu.VMEM((N_DEV, SQ_PER, DM), jnp.bfloat16),
            pltpu.SemaphoreType.DMA((N_DEV - 1,)),
            pltpu.SemaphoreType.DMA((N_DEV - 1,)),
        ],
        compiler_params=_CompilerParams(collective_id=0),
    )(x2, Wq_b)


NGLOB = 32


def _window_bias():
    qi = jnp.arange(SQ)[:, None]
    g_of_q = qi // SQ_PER
    w0 = jnp.maximum(0, g_of_q - 1) * SQ_PER
    ki0 = jnp.arange(SQ_PER)[None, :]
    m0 = ((jnp.abs(qi - ki0) <= 128) | (ki0 < NGLOB)) & (g_of_q >= 2)
    ki1 = w0 + jnp.arange(WIN)[None, :]
    m1 = (jnp.abs(qi - ki1) <= 128) | (ki1 < NGLOB)
    mask = jnp.concatenate([m0, m1], axis=1)
    return jnp.where(mask, 0.0, -1e9).astype(jnp.bfloat16)


def _attention(Q, K2, V2, bias):

    def body(q_ref, k_ref, v_ref, b_ref, o_ref):
        g = pl.program_id(1)
        qv = q_ref[...]
        w0 = jnp.maximum(0, g - 1) * SQ_PER

        kw = k_ref[pl.ds(w0, WIN), :]
        k0 = k_ref[0:SQ_PER, :]
        s0 = _dot(qv, k0, ((1,), (1,)))
        s1 = _dot(qv, kw, ((1,), (1,)))
        s = jnp.concatenate([s0, s1], axis=1) * SCALE
        s = s + b_ref[...].astype(jnp.float32)
        m = jnp.max(s, axis=1, keepdims=True)
        e = jnp.exp(s - m)
        w = (e / jnp.sum(e, axis=1, keepdims=True)).astype(jnp.bfloat16)
        ctx = (_dot(w[:, :SQ_PER], v_ref[0:SQ_PER, :], ((1,), (0,)))
               + _dot(w[:, SQ_PER:], v_ref[pl.ds(w0, WIN), :],
                      ((1,), (0,))))
        o_ref[...] = ctx.astype(jnp.bfloat16)

        @pl.when(g == 0)
        def _():
            q32 = qv[0:NGLOB, :]
            s32 = _dot(q32, k_ref[...], ((1,), (1,))) * SCALE
            m32 = jnp.max(s32, axis=1, keepdims=True)
            e32 = jnp.exp(s32 - m32)
            w32 = (e32 / jnp.sum(e32, axis=1, keepdims=True)).astype(
                jnp.bfloat16)
            o_ref[0:NGLOB, :] = _dot(
                w32, v_ref[...], ((1,), (0,))).astype(jnp.bfloat16)

    return pl.pallas_call(
        body,
        grid=(HQ_PER, N_DEV),
        out_shape=jax.ShapeDtypeStruct((SQ, HD_PER), jnp.bfloat16),
        in_specs=[
            pl.BlockSpec((SQ_PER, DH), lambda h, g: (g, h)),
            pl.BlockSpec((SKV, DH), lambda h, g: (0, h)),
            pl.BlockSpec((SKV, DH), lambda h, g: (0, h)),
            pl.BlockSpec((SQ_PER, SQ_PER + WIN), lambda h, g: (g, 0)),
        ],
        out_specs=pl.BlockSpec((SQ_PER, DH), lambda h, g: (g, h)),
    )(Q, K2, V2, bias)


def _rs_oproj(ctx, Wo_b):

    def body(c_ref, wo_ref, out_ref, sbuf, rbuf, send_sems, recv_sems):
        p = lax.axis_index("i")
        left = lax.rem(p - 1 + N_DEV, N_DEV)
        right = lax.rem(p + 1, N_DEV)
        _neighbor_barrier(left, right)

        wo = wo_ref[...]
        c0 = lax.rem(p - 1 + N_DEV, N_DEV)
        sbuf[0] = _dot(
            c_ref[pl.ds(c0 * SQ_PER, SQ_PER), :], wo, ((1,), (0,))
        ).astype(jnp.bfloat16)

        for s in range(N_DEV - 1):
            rdma = pltpu.make_async_remote_copy(
                src_ref=sbuf.at[s],
                dst_ref=rbuf.at[s],
                send_sem=send_sems.at[s],
                recv_sem=recv_sems.at[s],
                device_id=(right,),
                device_id_type=_DeviceIdType.MESH,
            )
            rdma.start()
            rc = lax.rem(p - s - 2 + 2 * N_DEV, N_DEV)
            own = _dot(c_ref[pl.ds(rc * SQ_PER, SQ_PER), :], wo, ((1,), (0,)))
            rdma.wait()
            if s < N_DEV - 2:
                sbuf[s + 1] = (rbuf[s].astype(jnp.float32) + own).astype(
                    jnp.bfloat16)
            else:
                out_ref[...] = rbuf[s].astype(jnp.float32) + own

    return pl.pallas_call(
        body,
        out_shape=jax.ShapeDtypeStruct((SQ_PER, DM), jnp.float32),
        in_specs=[pl.BlockSpec(memory_space=pltpu.VMEM)] * 2,
        out_specs=pl.BlockSpec(memory_space=pltpu.VMEM),
        scratch_shapes=[
            pltpu.VMEM((N_DEV - 1, SQ_PER, DM), jnp.bfloat16),
            pltpu.VMEM((N_DEV - 1, SQ_PER, DM), jnp.bfloat16),
            pltpu.SemaphoreType.DMA((N_DEV - 1,)),
            pltpu.SemaphoreType.DMA((N_DEV - 1,)),
        ],
        compiler_params=_CompilerParams(collective_id=1),
    )(ctx, Wo_b)


def kernel(x, Wq, K_ext, V_ext, Wo):
    p = lax.axis_index("i")

    x2 = x[0].astype(jnp.bfloat16)
    Wq_b = Wq.astype(jnp.bfloat16)
    Wo_b = Wo.astype(jnp.bfloat16)
    K_h = lax.dynamic_slice_in_dim(K_ext[0], p * HQ_PER, HQ_PER, axis=1)
    V_h = lax.dynamic_slice_in_dim(V_ext[0], p * HQ_PER, HQ_PER, axis=1)
    K2 = K_h.astype(jnp.bfloat16).reshape(SKV, HD_PER)
    V2 = V_h.astype(jnp.bfloat16).reshape(SKV, HD_PER)

    Q = _ag_qproj(x2, Wq_b)
    ctx = _attention(Q, K2, V2, _window_bias())
    out = _rs_oproj(ctx, Wo_b)
    return out[None]


# device time: 214017 ns/iter; 1.2459x vs baseline; 1.0374x over previous
import jax
import jax.numpy as jnp
from jax import lax
from jax.experimental import pallas as pl
from jax.experimental.pallas import tpu as pltpu

N_DEV = 8
SQ_PER = 256
SQ = N_DEV * SQ_PER
SKV = 4096
HQ_PER = 8
DH = 128
DM = 1024
HD_PER = HQ_PER * DH
WIN = 512
GW = 128
SCALE = 0.08838834764831843

_DeviceIdType = getattr(pl, "DeviceIdType", None) or pltpu.DeviceIdType
_sem_signal = getattr(pl, "semaphore_signal", None) or pltpu.semaphore_signal
_sem_wait = getattr(pl, "semaphore_wait", None) or pltpu.semaphore_wait
_CompilerParams = getattr(pltpu, "CompilerParams", None) or pltpu.TPUCompilerParams


def _neighbor_barrier(left, right):
    barrier_sem = pltpu.get_barrier_semaphore()
    for nbr in (left, right):
        _sem_signal(
            barrier_sem, inc=1,
            device_id=(nbr,), device_id_type=_DeviceIdType.MESH,
        )
    _sem_wait(barrier_sem, 2)


def _dot(a, b, dims):
    return jax.lax.dot_general(a, b, (dims, ((), ())),
                               preferred_element_type=jnp.float32)


def _ag_qproj(x2, Wq_b):

    def body(x_ref, wq_ref, q_ref, comm_ref, send_sems, recv_sems):
        p = lax.axis_index("i")
        left = lax.rem(p - 1 + N_DEV, N_DEV)
        right = lax.rem(p + 1, N_DEV)
        _neighbor_barrier(left, right)

        wq = wq_ref[...]
        comm_ref[0] = x_ref[...]
        for h in range(N_DEV - 1):
            rdma = pltpu.make_async_remote_copy(
                src_ref=comm_ref.at[h],
                dst_ref=comm_ref.at[h + 1],
                send_sem=send_sems.at[h],
                recv_sem=recv_sems.at[h],
                device_id=(right,),
                device_id_type=_DeviceIdType.MESH,
            )
            rdma.start()
            origin = lax.rem(p - h + N_DEV, N_DEV)
            q_ref[pl.ds(origin * SQ_PER, SQ_PER), :] = _dot(
                comm_ref[h], wq, ((1,), (0,))).astype(jnp.bfloat16)
            rdma.wait()
        origin = lax.rem(p + 1, N_DEV)
        q_ref[pl.ds(origin * SQ_PER, SQ_PER), :] = _dot(
            comm_ref[N_DEV - 1], wq, ((1,), (0,))).astype(jnp.bfloat16)

    return pl.pallas_call(
        body,
        out_shape=jax.ShapeDtypeStruct((SQ, HD_PER), jnp.bfloat16),
        in_specs=[pl.BlockSpec(memory_space=pltpu.VMEM)] * 2,
        out_specs=pl.BlockSpec(memory_space=pltpu.VMEM),
        scratch_shapes=[
            pltpu.VMEM((N_DEV, SQ_PER, DM), jnp.bfloat16),
            pltpu.SemaphoreType.DMA((N_DEV - 1,)),
            pltpu.SemaphoreType.DMA((N_DEV - 1,)),
        ],
        compiler_params=_CompilerParams(collective_id=0),
    )(x2, Wq_b)


NGLOB = 32


def _window_bias():
    qi = jnp.arange(SQ)[:, None]
    g_of_q = qi // SQ_PER
    w0 = jnp.maximum(0, g_of_q * SQ_PER - 128)
    ki0 = jnp.arange(GW)[None, :]
    m0 = (ki0 < NGLOB) & (g_of_q >= 1)
    ki1 = w0 + jnp.arange(WIN)[None, :]
    m1 = (jnp.abs(qi - ki1) <= 128) | (ki1 < NGLOB)
    mask = jnp.concatenate([m0, m1], axis=1)
    return jnp.where(mask, 0.0, -1e9).astype(jnp.bfloat16)


def _attention(Q, K2, V2, bias):

    def body(q_ref, k_ref, v_ref, b_ref, o_ref):
        g = pl.program_id(1)
        qv = q_ref[...]
        w0 = 128 * jnp.maximum(0, 2 * g - 1)

        kw = k_ref[pl.ds(w0, WIN), :]
        k0 = k_ref[0:GW, :]
        s0 = _dot(qv, k0, ((1,), (1,)))
        s1 = _dot(qv, kw, ((1,), (1,)))
        s = jnp.concatenate([s0, s1], axis=1) * SCALE
        s = s + b_ref[...].astype(jnp.float32)
        m = jnp.max(s, axis=1, keepdims=True)
        e = jnp.exp(s - m)
        w = (e / jnp.sum(e, axis=1, keepdims=True)).astype(jnp.bfloat16)
        ctx = (_dot(w[:, :GW], v_ref[0:GW, :], ((1,), (0,)))
               + _dot(w[:, GW:], v_ref[pl.ds(w0, WIN), :],
                      ((1,), (0,))))
        o_ref[...] = ctx.astype(jnp.bfloat16)

        @pl.when(g == 0)
        def _():
            q32 = qv[0:NGLOB, :]
            s32 = _dot(q32, k_ref[...], ((1,), (1,))) * SCALE
            m32 = jnp.max(s32, axis=1, keepdims=True)
            e32 = jnp.exp(s32 - m32)
            w32 = (e32 / jnp.sum(e32, axis=1, keepdims=True)).astype(
                jnp.bfloat16)
            o_ref[0:NGLOB, :] = _dot(
                w32, v_ref[...], ((1,), (0,))).astype(jnp.bfloat16)

    return pl.pallas_call(
        body,
        grid=(HQ_PER, N_DEV),
        out_shape=jax.ShapeDtypeStruct((SQ, HD_PER), jnp.bfloat16),
        in_specs=[
            pl.BlockSpec((SQ_PER, DH), lambda h, g: (g, h)),
            pl.BlockSpec((SKV, DH), lambda h, g: (0, h)),
            pl.BlockSpec((SKV, DH), lambda h, g: (0, h)),
            pl.BlockSpec((SQ_PER, GW + WIN), lambda h, g: (g, 0)),
        ],
        out_specs=pl.BlockSpec((SQ_PER, DH), lambda h, g: (g, h)),
    )(Q, K2, V2, bias)


def _rs_oproj(ctx, Wo_b):

    def body(c_ref, wo_ref, out_ref, sbuf, rbuf, send_sems, recv_sems):
        p = lax.axis_index("i")
        left = lax.rem(p - 1 + N_DEV, N_DEV)
        right = lax.rem(p + 1, N_DEV)
        _neighbor_barrier(left, right)

        wo = wo_ref[...]
        c0 = lax.rem(p - 1 + N_DEV, N_DEV)
        sbuf[0] = _dot(
            c_ref[pl.ds(c0 * SQ_PER, SQ_PER), :], wo, ((1,), (0,))
        ).astype(jnp.bfloat16)

        for s in range(N_DEV - 1):
            rdma = pltpu.make_async_remote_copy(
                src_ref=sbuf.at[s],
                dst_ref=rbuf.at[s],
                send_sem=send_sems.at[s],
                recv_sem=recv_sems.at[s],
                device_id=(right,),
                device_id_type=_DeviceIdType.MESH,
            )
            rdma.start()
            rc = lax.rem(p - s - 2 + 2 * N_DEV, N_DEV)
            own = _dot(c_ref[pl.ds(rc * SQ_PER, SQ_PER), :], wo, ((1,), (0,)))
            rdma.wait()
            if s < N_DEV - 2:
                sbuf[s + 1] = (rbuf[s].astype(jnp.float32) + own).astype(
                    jnp.bfloat16)
            else:
                out_ref[...] = rbuf[s].astype(jnp.float32) + own

    return pl.pallas_call(
        body,
        out_shape=jax.ShapeDtypeStruct((SQ_PER, DM), jnp.float32),
        in_specs=[pl.BlockSpec(memory_space=pltpu.VMEM)] * 2,
        out_specs=pl.BlockSpec(memory_space=pltpu.VMEM),
        scratch_shapes=[
            pltpu.VMEM((N_DEV - 1, SQ_PER, DM), jnp.bfloat16),
            pltpu.VMEM((N_DEV - 1, SQ_PER, DM), jnp.bfloat16),
            pltpu.SemaphoreType.DMA((N_DEV - 1,)),
            pltpu.SemaphoreType.DMA((N_DEV - 1,)),
        ],
        compiler_params=_CompilerParams(collective_id=1),
    )(ctx, Wo_b)


def kernel(x, Wq, K_ext, V_ext, Wo):
    p = lax.axis_index("i")

    x2 = x[0].astype(jnp.bfloat16)
    Wq_b = Wq.astype(jnp.bfloat16)
    Wo_b = Wo.astype(jnp.bfloat16)
    K_h = lax.dynamic_slice_in_dim(K_ext[0], p * HQ_PER, HQ_PER, axis=1)
    V_h = lax.dynamic_slice_in_dim(V_ext[0], p * HQ_PER, HQ_PER, axis=1)
    K2 = K_h.astype(jnp.bfloat16).reshape(SKV, HD_PER)
    V2 = V_h.astype(jnp.bfloat16).reshape(SKV, HD_PER)

    Q = _ag_qproj(x2, Wq_b)
    ctx = _attention(Q, K2, V2, _window_bias())
    out = _rs_oproj(ctx, Wo_b)
    return out[None]


# device time: 171743 ns/iter; 1.5526x vs baseline; 1.2461x over previous
import jax
import jax.numpy as jnp
from jax import lax
from jax.experimental import pallas as pl
from jax.experimental.pallas import tpu as pltpu

N_DEV = 8
SQ_PER = 256
SQ = N_DEV * SQ_PER
SKV = 4096
HQ_PER = 8
DH = 128
DM = 1024
HD_PER = HQ_PER * DH
WIN = 512
GW = 128
SCALE = 0.08838834764831843

_DeviceIdType = getattr(pl, "DeviceIdType", None) or pltpu.DeviceIdType
_sem_signal = getattr(pl, "semaphore_signal", None) or pltpu.semaphore_signal
_sem_wait = getattr(pl, "semaphore_wait", None) or pltpu.semaphore_wait
_CompilerParams = getattr(pltpu, "CompilerParams", None) or pltpu.TPUCompilerParams


def _neighbor_barrier(left, right):
    barrier_sem = pltpu.get_barrier_semaphore()
    for nbr in (left, right):
        _sem_signal(
            barrier_sem, inc=1,
            device_id=(nbr,), device_id_type=_DeviceIdType.MESH,
        )
    _sem_wait(barrier_sem, 2)


def _dot(a, b, dims):
    return jax.lax.dot_general(a, b, (dims, ((), ())),
                               preferred_element_type=jnp.float32)


def _ag_qproj(x2, Wq_b):

    def body(x_ref, wq_ref, q_ref, cw_ref, ccw_ref,
             cw_ssem, cw_rsem, ccw_ssem, ccw_rsem):
        p = lax.axis_index("i")
        left = lax.rem(p - 1 + N_DEV, N_DEV)
        right = lax.rem(p + 1, N_DEV)
        _neighbor_barrier(left, right)

        wq = wq_ref[...]

        def store_q(origin, chunk):
            q_ref[pl.ds(lax.rem(origin + 2 * N_DEV, N_DEV) * SQ_PER,
                        SQ_PER), :] = _dot(
                chunk, wq, ((1,), (0,))).astype(jnp.bfloat16)

        cw_ref[0] = x_ref[...]
        ccw_ref[0] = x_ref[...]
        for s in range(4):
            cw = pltpu.make_async_remote_copy(
                src_ref=cw_ref.at[s], dst_ref=cw_ref.at[s + 1],
                send_sem=cw_ssem.at[s], recv_sem=cw_rsem.at[s],
                device_id=(right,), device_id_type=_DeviceIdType.MESH,
            )
            cw.start()
            if s < 3:
                ccw = pltpu.make_async_remote_copy(
                    src_ref=ccw_ref.at[s], dst_ref=ccw_ref.at[s + 1],
                    send_sem=ccw_ssem.at[s], recv_sem=ccw_rsem.at[s],
                    device_id=(left,), device_id_type=_DeviceIdType.MESH,
                )
                ccw.start()
            if s == 0:
                store_q(p, cw_ref[0])
            else:
                store_q(p - s, cw_ref[s])
                store_q(p + s, ccw_ref[s])
            cw.wait()
            if s < 3:
                ccw.wait()
        store_q(p - 4, cw_ref[4])

    return pl.pallas_call(
        body,
        out_shape=jax.ShapeDtypeStruct((SQ, HD_PER), jnp.bfloat16),
        in_specs=[pl.BlockSpec(memory_space=pltpu.VMEM)] * 2,
        out_specs=pl.BlockSpec(memory_space=pltpu.VMEM),
        scratch_shapes=[
            pltpu.VMEM((5, SQ_PER, DM), jnp.bfloat16),
            pltpu.VMEM((4, SQ_PER, DM), jnp.bfloat16),
            pltpu.SemaphoreType.DMA((4,)),
            pltpu.SemaphoreType.DMA((4,)),
            pltpu.SemaphoreType.DMA((3,)),
            pltpu.SemaphoreType.DMA((3,)),
        ],
        compiler_params=_CompilerParams(collective_id=0),
    )(x2, Wq_b)


NGLOB = 32


def _window_bias():
    qi = jnp.arange(SQ)[:, None]
    g_of_q = qi // SQ_PER
    w0 = jnp.maximum(0, g_of_q * SQ_PER - 128)
    ki0 = jnp.arange(GW)[None, :]
    m0 = (ki0 < NGLOB) & (g_of_q >= 1)
    ki1 = w0 + jnp.arange(WIN)[None, :]
    m1 = (jnp.abs(qi - ki1) <= 128) | (ki1 < NGLOB)
    mask = jnp.concatenate([m0, m1], axis=1)
    return jnp.where(mask, 0.0, -1e9).astype(jnp.bfloat16)


def _attention(Q, K2, V2, bias):

    def body(q_ref, k_ref, v_ref, b_ref, o_ref):
        g = pl.program_id(1)
        qv = q_ref[...]
        w0 = 128 * jnp.maximum(0, 2 * g - 1)

        kw = k_ref[pl.ds(w0, WIN), :]
        k0 = k_ref[0:GW, :]
        s0 = _dot(qv, k0, ((1,), (1,)))
        s1 = _dot(qv, kw, ((1,), (1,)))
        s = jnp.concatenate([s0, s1], axis=1) * SCALE
        s = s + b_ref[...].astype(jnp.float32)
        m = jnp.max(s, axis=1, keepdims=True)
        e = jnp.exp(s - m)
        w = (e / jnp.sum(e, axis=1, keepdims=True)).astype(jnp.bfloat16)
        ctx = (_dot(w[:, :GW], v_ref[0:GW, :], ((1,), (0,)))
               + _dot(w[:, GW:], v_ref[pl.ds(w0, WIN), :],
                      ((1,), (0,))))
        o_ref[...] = ctx.astype(jnp.bfloat16)

        @pl.when(g == 0)
        def _():
            q32 = qv[0:NGLOB, :]
            s32 = _dot(q32, k_ref[...], ((1,), (1,))) * SCALE
            m32 = jnp.max(s32, axis=1, keepdims=True)
            e32 = jnp.exp(s32 - m32)
            w32 = (e32 / jnp.sum(e32, axis=1, keepdims=True)).astype(
                jnp.bfloat16)
            o_ref[0:NGLOB, :] = _dot(
                w32, v_ref[...], ((1,), (0,))).astype(jnp.bfloat16)

    return pl.pallas_call(
        body,
        grid=(HQ_PER, N_DEV),
        out_shape=jax.ShapeDtypeStruct((SQ, HD_PER), jnp.bfloat16),
        in_specs=[
            pl.BlockSpec((SQ_PER, DH), lambda h, g: (g, h)),
            pl.BlockSpec((SKV, DH), lambda h, g: (0, h)),
            pl.BlockSpec((SKV, DH), lambda h, g: (0, h)),
            pl.BlockSpec((SQ_PER, GW + WIN), lambda h, g: (g, 0)),
        ],
        out_specs=pl.BlockSpec((SQ_PER, DH), lambda h, g: (g, h)),
    )(Q, K2, V2, bias)


def _rs_oproj(ctx, Wo_b):

    def body(c_ref, wo_ref, out_ref, cw_s, cw_r, ccw_s, ccw_r,
             cw_ssem, cw_rsem, ccw_ssem, ccw_rsem):
        p = lax.axis_index("i")
        left = lax.rem(p - 1 + N_DEV, N_DEV)
        right = lax.rem(p + 1, N_DEV)
        _neighbor_barrier(left, right)

        wo = wo_ref[...]

        def partial(c):
            c = lax.rem(c + 2 * N_DEV, N_DEV)
            return _dot(c_ref[pl.ds(c * SQ_PER, SQ_PER), :], wo, ((1,), (0,)))

        cw_s[0] = partial(p + 4).astype(jnp.bfloat16)
        ccw_s[0] = partial(p - 3).astype(jnp.bfloat16)
        own_last = None
        for s in range(4):
            cw = pltpu.make_async_remote_copy(
                src_ref=cw_s.at[s], dst_ref=cw_r.at[s],
                send_sem=cw_ssem.at[s], recv_sem=cw_rsem.at[s],
                device_id=(right,), device_id_type=_DeviceIdType.MESH,
            )
            cw.start()
            if s < 3:
                ccw = pltpu.make_async_remote_copy(
                    src_ref=ccw_s.at[s], dst_ref=ccw_r.at[s],
                    send_sem=ccw_ssem.at[s], recv_sem=ccw_rsem.at[s],
                    device_id=(left,), device_id_type=_DeviceIdType.MESH,
                )
                ccw.start()
            nxt_cw = partial(p + 3 - s) if s < 3 else None
            nxt_ccw = partial(p - 2 + s) if s < 2 else None
            if s == 3:
                own_last = partial(p)
            cw.wait()
            if s < 3:
                ccw.wait()
                cw_s[s + 1] = (cw_r[s].astype(jnp.float32)
                               + nxt_cw).astype(jnp.bfloat16)
            if s < 2:
                ccw_s[s + 1] = (ccw_r[s].astype(jnp.float32)
                                + nxt_ccw).astype(jnp.bfloat16)
        out_ref[...] = (cw_r[3].astype(jnp.float32)
                        + ccw_r[2].astype(jnp.float32) + own_last)

    return pl.pallas_call(
        body,
        out_shape=jax.ShapeDtypeStruct((SQ_PER, DM), jnp.float32),
        in_specs=[pl.BlockSpec(memory_space=pltpu.VMEM)] * 2,
        out_specs=pl.BlockSpec(memory_space=pltpu.VMEM),
        scratch_shapes=[
            pltpu.VMEM((4, SQ_PER, DM), jnp.bfloat16),
            pltpu.VMEM((4, SQ_PER, DM), jnp.bfloat16),
            pltpu.VMEM((3, SQ_PER, DM), jnp.bfloat16),
            pltpu.VMEM((3, SQ_PER, DM), jnp.bfloat16),
            pltpu.SemaphoreType.DMA((4,)),
            pltpu.SemaphoreType.DMA((4,)),
            pltpu.SemaphoreType.DMA((3,)),
            pltpu.SemaphoreType.DMA((3,)),
        ],
        compiler_params=_CompilerParams(collective_id=1),
    )(ctx, Wo_b)


def kernel(x, Wq, K_ext, V_ext, Wo):
    p = lax.axis_index("i")

    x2 = x[0].astype(jnp.bfloat16)
    Wq_b = Wq.astype(jnp.bfloat16)
    Wo_b = Wo.astype(jnp.bfloat16)
    K_h = lax.dynamic_slice_in_dim(K_ext[0], p * HQ_PER, HQ_PER, axis=1)
    V_h = lax.dynamic_slice_in_dim(V_ext[0], p * HQ_PER, HQ_PER, axis=1)
    K2 = K_h.astype(jnp.bfloat16).reshape(SKV, HD_PER)
    V2 = V_h.astype(jnp.bfloat16).reshape(SKV, HD_PER)

    Q = _ag_qproj(x2, Wq_b)
    ctx = _attention(Q, K2, V2, _window_bias())
    out = _rs_oproj(ctx, Wo_b)
    return out[None]


# device time: 153421 ns/iter; 1.7380x vs baseline; 1.1194x over previous
import jax
import jax.numpy as jnp
from jax import lax
from jax.experimental import pallas as pl
from jax.experimental.pallas import tpu as pltpu

N_DEV = 8
SQ_PER = 256
SQ = N_DEV * SQ_PER
SKV = 4096
HQ_PER = 8
DH = 128
DM = 1024
HD_PER = HQ_PER * DH
WIN = 512
GW = 128
SCALE = 0.08838834764831843

_DeviceIdType = getattr(pl, "DeviceIdType", None) or pltpu.DeviceIdType
_sem_signal = getattr(pl, "semaphore_signal", None) or pltpu.semaphore_signal
_sem_wait = getattr(pl, "semaphore_wait", None) or pltpu.semaphore_wait
_CompilerParams = getattr(pltpu, "CompilerParams", None) or pltpu.TPUCompilerParams


def _neighbor_barrier(left, right):
    barrier_sem = pltpu.get_barrier_semaphore()
    for nbr in (left, right):
        _sem_signal(
            barrier_sem, inc=1,
            device_id=(nbr,), device_id_type=_DeviceIdType.MESH,
        )
    _sem_wait(barrier_sem, 2)


def _dot(a, b, dims):
    return jax.lax.dot_general(a, b, (dims, ((), ())),
                               preferred_element_type=jnp.float32)


def _ag_qproj(x2, Wq_b):

    def body(x_ref, wq_ref, q_ref, cw_ref, ccw_ref,
             cw_ssem, cw_rsem, ccw_ssem, ccw_rsem):
        p = lax.axis_index("i")
        left = lax.rem(p - 1 + N_DEV, N_DEV)
        right = lax.rem(p + 1, N_DEV)
        _neighbor_barrier(left, right)

        wq = wq_ref[...]

        def store_q(origin, chunk):
            q_ref[pl.ds(lax.rem(origin + 2 * N_DEV, N_DEV) * SQ_PER,
                        SQ_PER), :] = _dot(
                chunk, wq, ((1,), (0,))).astype(jnp.bfloat16)

        cw_ref[0] = x_ref[...]
        ccw_ref[0] = x_ref[...]
        for s in range(4):
            cw = pltpu.make_async_remote_copy(
                src_ref=cw_ref.at[s], dst_ref=cw_ref.at[s + 1],
                send_sem=cw_ssem.at[s], recv_sem=cw_rsem.at[s],
                device_id=(right,), device_id_type=_DeviceIdType.MESH,
            )
            cw.start()
            if s < 3:
                ccw = pltpu.make_async_remote_copy(
                    src_ref=ccw_ref.at[s], dst_ref=ccw_ref.at[s + 1],
                    send_sem=ccw_ssem.at[s], recv_sem=ccw_rsem.at[s],
                    device_id=(left,), device_id_type=_DeviceIdType.MESH,
                )
                ccw.start()
            if s == 0:
                store_q(p, cw_ref[0])
            else:
                store_q(p - s, cw_ref[s])
                store_q(p + s, ccw_ref[s])
            cw.wait()
            if s < 3:
                ccw.wait()
        store_q(p - 4, cw_ref[4])

    return pl.pallas_call(
        body,
        out_shape=jax.ShapeDtypeStruct((SQ, HD_PER), jnp.bfloat16),
        in_specs=[pl.BlockSpec(memory_space=pltpu.VMEM)] * 2,
        out_specs=pl.BlockSpec(memory_space=pltpu.VMEM),
        scratch_shapes=[
            pltpu.VMEM((5, SQ_PER, DM), jnp.bfloat16),
            pltpu.VMEM((4, SQ_PER, DM), jnp.bfloat16),
            pltpu.SemaphoreType.DMA((4,)),
            pltpu.SemaphoreType.DMA((4,)),
            pltpu.SemaphoreType.DMA((3,)),
            pltpu.SemaphoreType.DMA((3,)),
        ],
        compiler_params=_CompilerParams(collective_id=0),
    )(x2, Wq_b)


NGLOB = 32


def _window_bias():
    qi = jnp.arange(SQ)[:, None]
    g_of_q = qi // SQ_PER
    w0 = jnp.maximum(0, g_of_q * SQ_PER - 128)
    ki0 = jnp.arange(GW)[None, :]
    m0 = (ki0 < NGLOB) & (g_of_q >= 1)
    ki1 = w0 + jnp.arange(WIN)[None, :]
    m1 = (jnp.abs(qi - ki1) <= 128) | (ki1 < NGLOB)
    mask = jnp.concatenate([m0, m1], axis=1)
    return jnp.where(mask, 0.0, -1e9).astype(jnp.bfloat16)


def _attention(Q, K2, V2, bias):

    def body(q_ref, k_ref, v_ref, b_ref, o_ref):
        g = pl.program_id(1)
        qv = q_ref[...]
        w0 = 128 * jnp.maximum(0, 2 * g - 1)

        kw = k_ref[pl.ds(w0, WIN), :]
        k0 = k_ref[0:GW, :]
        s0 = _dot(qv, k0, ((1,), (1,)))
        s1 = _dot(qv, kw, ((1,), (1,)))
        s = jnp.concatenate([s0, s1], axis=1) * SCALE
        s = s + b_ref[...].astype(jnp.float32)
        e = jnp.exp(s).astype(jnp.bfloat16)
        denom = jnp.sum(e.astype(jnp.float32), axis=1, keepdims=True)
        ctx = (_dot(e[:, :GW], v_ref[0:GW, :], ((1,), (0,)))
               + _dot(e[:, GW:], v_ref[pl.ds(w0, WIN), :],
                      ((1,), (0,))))
        o_ref[...] = (ctx / denom).astype(jnp.bfloat16)

        @pl.when(g == 0)
        def _():
            q32 = qv[0:NGLOB, :]
            s32 = _dot(q32, k_ref[...], ((1,), (1,))) * SCALE
            e32 = jnp.exp(s32).astype(jnp.bfloat16)
            d32 = jnp.sum(e32.astype(jnp.float32), axis=1, keepdims=True)
            ctx32 = _dot(e32, v_ref[...], ((1,), (0,)))
            o_ref[0:NGLOB, :] = (ctx32 / d32).astype(jnp.bfloat16)

    return pl.pallas_call(
        body,
        grid=(HQ_PER, N_DEV),
        out_shape=jax.ShapeDtypeStruct((SQ, HD_PER), jnp.bfloat16),
        in_specs=[
            pl.BlockSpec((SQ_PER, DH), lambda h, g: (g, h)),
            pl.BlockSpec((SKV, DH), lambda h, g: (0, h)),
            pl.BlockSpec((SKV, DH), lambda h, g: (0, h)),
            pl.BlockSpec((SQ_PER, GW + WIN), lambda h, g: (g, 0)),
        ],
        out_specs=pl.BlockSpec((SQ_PER, DH), lambda h, g: (g, h)),
    )(Q, K2, V2, bias)


def _rs_oproj(ctx, Wo_b):

    def body(c_ref, wo_ref, out_ref, cw_s, cw_r, ccw_s, ccw_r,
             cw_ssem, cw_rsem, ccw_ssem, ccw_rsem):
        p = lax.axis_index("i")
        left = lax.rem(p - 1 + N_DEV, N_DEV)
        right = lax.rem(p + 1, N_DEV)
        _neighbor_barrier(left, right)

        wo = wo_ref[...]

        def partial(c):
            c = lax.rem(c + 2 * N_DEV, N_DEV)
            return _dot(c_ref[pl.ds(c * SQ_PER, SQ_PER), :], wo, ((1,), (0,)))

        cw_s[0] = partial(p + 4).astype(jnp.bfloat16)
        ccw_s[0] = partial(p - 3).astype(jnp.bfloat16)
        own_last = None
        for s in range(4):
            cw = pltpu.make_async_remote_copy(
                src_ref=cw_s.at[s], dst_ref=cw_r.at[s],
                send_sem=cw_ssem.at[s], recv_sem=cw_rsem.at[s],
                device_id=(right,), device_id_type=_DeviceIdType.MESH,
            )
            cw.start()
            if s < 3:
                ccw = pltpu.make_async_remote_copy(
                    src_ref=ccw_s.at[s], dst_ref=ccw_r.at[s],
                    send_sem=ccw_ssem.at[s], recv_sem=ccw_rsem.at[s],
                    device_id=(left,), device_id_type=_DeviceIdType.MESH,
                )
                ccw.start()
            nxt_cw = partial(p + 3 - s) if s < 3 else None
            nxt_ccw = partial(p - 2 + s) if s < 2 else None
            if s == 3:
                own_last = partial(p)
            cw.wait()
            if s < 3:
                ccw.wait()
                cw_s[s + 1] = (cw_r[s].astype(jnp.float32)
                               + nxt_cw).astype(jnp.bfloat16)
            if s < 2:
                ccw_s[s + 1] = (ccw_r[s].astype(jnp.float32)
                                + nxt_ccw).astype(jnp.bfloat16)
        out_ref[...] = (cw_r[3].astype(jnp.float32)
                        + ccw_r[2].astype(jnp.float32) + own_last)

    return pl.pallas_call(
        body,
        out_shape=jax.ShapeDtypeStruct((SQ_PER, DM), jnp.float32),
        in_specs=[pl.BlockSpec(memory_space=pltpu.VMEM)] * 2,
        out_specs=pl.BlockSpec(memory_space=pltpu.VMEM),
        scratch_shapes=[
            pltpu.VMEM((4, SQ_PER, DM), jnp.bfloat16),
            pltpu.VMEM((4, SQ_PER, DM), jnp.bfloat16),
            pltpu.VMEM((3, SQ_PER, DM), jnp.bfloat16),
            pltpu.VMEM((3, SQ_PER, DM), jnp.bfloat16),
            pltpu.SemaphoreType.DMA((4,)),
            pltpu.SemaphoreType.DMA((4,)),
            pltpu.SemaphoreType.DMA((3,)),
            pltpu.SemaphoreType.DMA((3,)),
        ],
        compiler_params=_CompilerParams(collective_id=1),
    )(ctx, Wo_b)


def kernel(x, Wq, K_ext, V_ext, Wo):
    p = lax.axis_index("i")

    x2 = x[0].astype(jnp.bfloat16)
    Wq_b = Wq.astype(jnp.bfloat16)
    Wo_b = Wo.astype(jnp.bfloat16)
    K_h = lax.dynamic_slice_in_dim(K_ext[0], p * HQ_PER, HQ_PER, axis=1)
    V_h = lax.dynamic_slice_in_dim(V_ext[0], p * HQ_PER, HQ_PER, axis=1)
    K2 = K_h.astype(jnp.bfloat16).reshape(SKV, HD_PER)
    V2 = V_h.astype(jnp.bfloat16).reshape(SKV, HD_PER)

    Q = _ag_qproj(x2, Wq_b)
    ctx = _attention(Q, K2, V2, _window_bias())
    out = _rs_oproj(ctx, Wo_b)
    return out[None]


# device time: 137338 ns/iter; 1.9416x vs baseline; 1.1171x over previous
import jax
import jax.numpy as jnp
from jax import lax
from jax.experimental import pallas as pl
from jax.experimental.pallas import tpu as pltpu

N_DEV = 8
SQ_PER = 256
SQ = N_DEV * SQ_PER
SKV = 4096
HQ_PER = 8
DH = 128
DM = 1024
HD_PER = HQ_PER * DH
WIN = 512
GW = 128
SCALE = 0.08838834764831843

_DeviceIdType = getattr(pl, "DeviceIdType", None) or pltpu.DeviceIdType
_sem_signal = getattr(pl, "semaphore_signal", None) or pltpu.semaphore_signal
_sem_wait = getattr(pl, "semaphore_wait", None) or pltpu.semaphore_wait
_CompilerParams = getattr(pltpu, "CompilerParams", None) or pltpu.TPUCompilerParams


def _neighbor_barrier(left, right):
    barrier_sem = pltpu.get_barrier_semaphore()
    for nbr in (left, right):
        _sem_signal(
            barrier_sem, inc=1,
            device_id=(nbr,), device_id_type=_DeviceIdType.MESH,
        )
    _sem_wait(barrier_sem, 2)


def _dot(a, b, dims):
    return jax.lax.dot_general(a, b, (dims, ((), ())),
                               preferred_element_type=jnp.float32)


def _ag_qproj(x2, Wq_b):

    def body(x_ref, wq_ref, q_ref, cw_ref, ccw_ref,
             cw_ssem, cw_rsem, ccw_ssem, ccw_rsem):
        p = lax.axis_index("i")
        left = lax.rem(p - 1 + N_DEV, N_DEV)
        right = lax.rem(p + 1, N_DEV)
        _neighbor_barrier(left, right)

        wq = wq_ref[...]

        def store_q(origin, chunk):
            q_ref[pl.ds(lax.rem(origin + 2 * N_DEV, N_DEV) * SQ_PER,
                        SQ_PER), :] = _dot(
                chunk, wq, ((1,), (0,))).astype(jnp.bfloat16)

        cw_ref[0] = x_ref[...]
        ccw_ref[0] = x_ref[...]
        for s in range(4):
            cw = pltpu.make_async_remote_copy(
                src_ref=cw_ref.at[s], dst_ref=cw_ref.at[s + 1],
                send_sem=cw_ssem.at[s], recv_sem=cw_rsem.at[s],
                device_id=(right,), device_id_type=_DeviceIdType.MESH,
            )
            cw.start()
            if s < 3:
                ccw = pltpu.make_async_remote_copy(
                    src_ref=ccw_ref.at[s], dst_ref=ccw_ref.at[s + 1],
                    send_sem=ccw_ssem.at[s], recv_sem=ccw_rsem.at[s],
                    device_id=(left,), device_id_type=_DeviceIdType.MESH,
                )
                ccw.start()
            if s == 0:
                store_q(p, cw_ref[0])
            else:
                store_q(p - s, cw_ref[s])
                store_q(p + s, ccw_ref[s])
            cw.wait()
            if s < 3:
                ccw.wait()
        store_q(p - 4, cw_ref[4])

    return pl.pallas_call(
        body,
        out_shape=jax.ShapeDtypeStruct((SQ, HD_PER), jnp.bfloat16),
        in_specs=[pl.BlockSpec(memory_space=pltpu.VMEM)] * 2,
        out_specs=pl.BlockSpec(memory_space=pltpu.VMEM),
        scratch_shapes=[
            pltpu.VMEM((5, SQ_PER, DM), jnp.bfloat16),
            pltpu.VMEM((4, SQ_PER, DM), jnp.bfloat16),
            pltpu.SemaphoreType.DMA((4,)),
            pltpu.SemaphoreType.DMA((4,)),
            pltpu.SemaphoreType.DMA((3,)),
            pltpu.SemaphoreType.DMA((3,)),
        ],
        compiler_params=_CompilerParams(collective_id=0),
    )(x2, Wq_b)


NGLOB = 32


def _window_bias():
    qi = jnp.arange(SQ)[:, None]
    g_of_q = qi // SQ_PER
    w0 = jnp.maximum(0, g_of_q * SQ_PER - 128)
    ki0 = jnp.arange(GW)[None, :]
    m0 = (ki0 < NGLOB) & (g_of_q >= 1)
    ki1 = w0 + jnp.arange(WIN)[None, :]
    m1 = (jnp.abs(qi - ki1) <= 128) | (ki1 < NGLOB)
    mask = jnp.concatenate([m0, m1], axis=1)
    return jnp.where(mask, 0.0, -1e9).astype(jnp.bfloat16)


def _attention(Q, K_f32, V_f32, bias):
    def body(q_ref, kany_ref, vany_ref, b_ref, o_ref,
             kf32, vf32, k_ref, v_ref, dsems):
        h = pl.program_id(0)
        g = pl.program_id(1)

        @pl.when(g == 0)
        def _():
            head = lax.axis_index("i") * HQ_PER + h
            kcp = pltpu.make_async_copy(
                kany_ref.at[:, head, :], kf32, dsems.at[0])
            vcp = pltpu.make_async_copy(
                vany_ref.at[:, head, :], vf32, dsems.at[1])
            kcp.start()
            vcp.start()
            kcp.wait()
            k_ref[...] = kf32[...].astype(jnp.bfloat16)
            vcp.wait()
            v_ref[...] = vf32[...].astype(jnp.bfloat16)
        qv = q_ref[...]
        w0 = 128 * jnp.maximum(0, 2 * g - 1)

        kw = k_ref[pl.ds(w0, WIN), :]
        k0 = k_ref[0:GW, :]
        s0 = _dot(qv, k0, ((1,), (1,)))
        s1 = _dot(qv, kw, ((1,), (1,)))
        s = jnp.concatenate([s0, s1], axis=1) * SCALE
        s = s + b_ref[...].astype(jnp.float32)
        e = jnp.exp(s).astype(jnp.bfloat16)
        denom = jnp.sum(e.astype(jnp.float32), axis=1, keepdims=True)
        ctx = (_dot(e[:, :GW], v_ref[0:GW, :], ((1,), (0,)))
               + _dot(e[:, GW:], v_ref[pl.ds(w0, WIN), :],
                      ((1,), (0,))))
        o_ref[...] = (ctx / denom).astype(jnp.bfloat16)

        @pl.when(g == 0)
        def _():
            q32 = qv[0:NGLOB, :]
            s32 = _dot(q32, k_ref[...], ((1,), (1,))) * SCALE
            e32 = jnp.exp(s32).astype(jnp.bfloat16)
            d32 = jnp.sum(e32.astype(jnp.float32), axis=1, keepdims=True)
            ctx32 = _dot(e32, v_ref[...], ((1,), (0,)))
            o_ref[0:NGLOB, :] = (ctx32 / d32).astype(jnp.bfloat16)

    return pl.pallas_call(
        body,
        grid=(HQ_PER, N_DEV),
        out_shape=jax.ShapeDtypeStruct((SQ, HD_PER), jnp.bfloat16),
        in_specs=[
            pl.BlockSpec((SQ_PER, DH), lambda h, g: (g, h)),
            pl.BlockSpec(memory_space=pl.ANY),
            pl.BlockSpec(memory_space=pl.ANY),
            pl.BlockSpec((SQ_PER, GW + WIN), lambda h, g: (g, 0)),
        ],
        out_specs=pl.BlockSpec((SQ_PER, DH), lambda h, g: (g, h)),
        scratch_shapes=[
            pltpu.VMEM((SKV, DH), jnp.float32),
            pltpu.VMEM((SKV, DH), jnp.float32),
            pltpu.VMEM((SKV, DH), jnp.bfloat16),
            pltpu.VMEM((SKV, DH), jnp.bfloat16),
            pltpu.SemaphoreType.DMA((2,)),
        ],
    )(Q, K_f32, V_f32, bias)


def _rs_oproj(ctx, Wo_b):

    def body(c_ref, wo_ref, out_ref, cw_s, cw_r, ccw_s, ccw_r,
             cw_ssem, cw_rsem, ccw_ssem, ccw_rsem):
        p = lax.axis_index("i")
        left = lax.rem(p - 1 + N_DEV, N_DEV)
        right = lax.rem(p + 1, N_DEV)
        _neighbor_barrier(left, right)

        wo = wo_ref[...]

        def partial(c):
            c = lax.rem(c + 2 * N_DEV, N_DEV)
            return _dot(c_ref[pl.ds(c * SQ_PER, SQ_PER), :], wo, ((1,), (0,)))

        cw_s[0] = partial(p + 4).astype(jnp.bfloat16)
        ccw_s[0] = partial(p - 3).astype(jnp.bfloat16)
        own_last = None
        for s in range(4):
            cw = pltpu.make_async_remote_copy(
                src_ref=cw_s.at[s], dst_ref=cw_r.at[s],
                send_sem=cw_ssem.at[s], recv_sem=cw_rsem.at[s],
                device_id=(right,), device_id_type=_DeviceIdType.MESH,
            )
            cw.start()
            if s < 3:
                ccw = pltpu.make_async_remote_copy(
                    src_ref=ccw_s.at[s], dst_ref=ccw_r.at[s],
                    send_sem=ccw_ssem.at[s], recv_sem=ccw_rsem.at[s],
                    device_id=(left,), device_id_type=_DeviceIdType.MESH,
                )
                ccw.start()
            nxt_cw = partial(p + 3 - s) if s < 3 else None
            nxt_ccw = partial(p - 2 + s) if s < 2 else None
            if s == 3:
                own_last = partial(p)
            cw.wait()
            if s < 3:
                ccw.wait()
                cw_s[s + 1] = (cw_r[s].astype(jnp.float32)
                               + nxt_cw).astype(jnp.bfloat16)
            if s < 2:
                ccw_s[s + 1] = (ccw_r[s].astype(jnp.float32)
                                + nxt_ccw).astype(jnp.bfloat16)
        out_ref[...] = (cw_r[3].astype(jnp.float32)
                        + ccw_r[2].astype(jnp.float32) + own_last)

    return pl.pallas_call(
        body,
        out_shape=jax.ShapeDtypeStruct((SQ_PER, DM), jnp.float32),
        in_specs=[pl.BlockSpec(memory_space=pltpu.VMEM)] * 2,
        out_specs=pl.BlockSpec(memory_space=pltpu.VMEM),
        scratch_shapes=[
            pltpu.VMEM((4, SQ_PER, DM), jnp.bfloat16),
            pltpu.VMEM((4, SQ_PER, DM), jnp.bfloat16),
            pltpu.VMEM((3, SQ_PER, DM), jnp.bfloat16),
            pltpu.VMEM((3, SQ_PER, DM), jnp.bfloat16),
            pltpu.SemaphoreType.DMA((4,)),
            pltpu.SemaphoreType.DMA((4,)),
            pltpu.SemaphoreType.DMA((3,)),
            pltpu.SemaphoreType.DMA((3,)),
        ],
        compiler_params=_CompilerParams(collective_id=1),
    )(ctx, Wo_b)


def kernel(x, Wq, K_ext, V_ext, Wo):
    x2 = x[0].astype(jnp.bfloat16)
    Wq_b = Wq.astype(jnp.bfloat16)
    Wo_b = Wo.astype(jnp.bfloat16)

    Q = _ag_qproj(x2, Wq_b)
    ctx = _attention(Q, K_ext[0], V_ext[0], _window_bias())
    out = _rs_oproj(ctx, Wo_b)
    return out[None]


# device time: 121292 ns/iter; 2.1984x vs baseline; 1.1323x over previous
import jax
import jax.numpy as jnp
from jax import lax
from jax.experimental import pallas as pl
from jax.experimental.pallas import tpu as pltpu

N_DEV = 8
SQ_PER = 256
SQ = N_DEV * SQ_PER
SKV = 4096
HQ_PER = 8
DH = 128
DM = 1024
HD_PER = HQ_PER * DH
WIN = 512
GW = 128
SCALE = 0.08838834764831843

_DeviceIdType = getattr(pl, "DeviceIdType", None) or pltpu.DeviceIdType
_sem_signal = getattr(pl, "semaphore_signal", None) or pltpu.semaphore_signal
_sem_wait = getattr(pl, "semaphore_wait", None) or pltpu.semaphore_wait
_CompilerParams = getattr(pltpu, "CompilerParams", None) or pltpu.TPUCompilerParams


def _neighbor_barrier(left, right):
    barrier_sem = pltpu.get_barrier_semaphore()
    for nbr in (left, right):
        _sem_signal(
            barrier_sem, inc=1,
            device_id=(nbr,), device_id_type=_DeviceIdType.MESH,
        )
    _sem_wait(barrier_sem, 2)


def _dot(a, b, dims):
    return jax.lax.dot_general(a, b, (dims, ((), ())),
                               preferred_element_type=jnp.float32)


def _ag_qproj(x2, Wq_b):

    def body(x_ref, wq_ref, q_ref, cw_ref, ccw_ref,
             cw_ssem, cw_rsem, ccw_ssem, ccw_rsem):
        p = lax.axis_index("i")
        left = lax.rem(p - 1 + N_DEV, N_DEV)
        right = lax.rem(p + 1, N_DEV)
        _neighbor_barrier(left, right)

        wq = wq_ref[...]

        def store_q(origin, chunk):
            q_ref[pl.ds(lax.rem(origin + 2 * N_DEV, N_DEV) * SQ_PER,
                        SQ_PER), :] = _dot(
                chunk, wq, ((1,), (0,))).astype(jnp.bfloat16)

        cw_ref[0] = x_ref[...]
        ccw_ref[0] = x_ref[...]
        for s in range(4):
            cw = pltpu.make_async_remote_copy(
                src_ref=cw_ref.at[s], dst_ref=cw_ref.at[s + 1],
                send_sem=cw_ssem.at[s], recv_sem=cw_rsem.at[s],
                device_id=(right,), device_id_type=_DeviceIdType.MESH,
            )
            cw.start()
            if s < 3:
                ccw = pltpu.make_async_remote_copy(
                    src_ref=ccw_ref.at[s], dst_ref=ccw_ref.at[s + 1],
                    send_sem=ccw_ssem.at[s], recv_sem=ccw_rsem.at[s],
                    device_id=(left,), device_id_type=_DeviceIdType.MESH,
                )
                ccw.start()
            if s == 0:
                store_q(p, cw_ref[0])
            else:
                store_q(p - s, cw_ref[s])
                store_q(p + s, ccw_ref[s])
            cw.wait()
            if s < 3:
                ccw.wait()
        store_q(p - 4, cw_ref[4])

    return pl.pallas_call(
        body,
        out_shape=jax.ShapeDtypeStruct((SQ, HD_PER), jnp.bfloat16),
        in_specs=[pl.BlockSpec(memory_space=pltpu.VMEM)] * 2,
        out_specs=pl.BlockSpec(memory_space=pltpu.VMEM),
        scratch_shapes=[
            pltpu.VMEM((5, SQ_PER, DM), jnp.bfloat16),
            pltpu.VMEM((4, SQ_PER, DM), jnp.bfloat16),
            pltpu.SemaphoreType.DMA((4,)),
            pltpu.SemaphoreType.DMA((4,)),
            pltpu.SemaphoreType.DMA((3,)),
            pltpu.SemaphoreType.DMA((3,)),
        ],
        compiler_params=_CompilerParams(collective_id=0),
    )(x2, Wq_b)


NGLOB = 32


def _window_bias():
    qi = jnp.arange(SQ)[:, None]
    g_of_q = qi // SQ_PER
    w0 = jnp.maximum(0, g_of_q * SQ_PER - 128)
    ki0 = jnp.arange(GW)[None, :]
    m0 = (ki0 < NGLOB) & (g_of_q >= 1)
    ki1 = w0 + jnp.arange(WIN)[None, :]
    m1 = (jnp.abs(qi - ki1) <= 128) | (ki1 < NGLOB)
    mask = jnp.concatenate([m0, m1], axis=1)
    return jnp.where(mask, 0.0, -1e9).astype(jnp.bfloat16)


def _attention(Q, K_f32, V_f32, bias):
    def body(q_ref, kany_ref, vany_ref, b_ref, o_ref,
             kf32, vf32, k_ref, v_ref, ksems, vsems):
        h = pl.program_id(0)
        g = pl.program_id(1)

        def kv_copy(head, slot):
            return (
                pltpu.make_async_copy(
                    kany_ref.at[:, head, :], kf32.at[slot], ksems.at[slot]),
                pltpu.make_async_copy(
                    vany_ref.at[:, head, :], vf32.at[slot], vsems.at[slot]),
            )

        @pl.when(g == 0)
        def _():
            me = lax.axis_index("i") * HQ_PER
            slot = lax.rem(h, 2)

            @pl.when(h == 0)
            def _():
                kcp0, vcp0 = kv_copy(me, 0)
                kcp0.start()
                vcp0.start()

            kcp, vcp = kv_copy(me + h, slot)

            @pl.when(h < HQ_PER - 1)
            def _():
                kcpn, vcpn = kv_copy(me + h + 1, 1 - slot)
                kcpn.start()
                vcpn.start()

            kcp.wait()
            k_ref[...] = kf32[slot].astype(jnp.bfloat16)
            vcp.wait()
            v_ref[...] = vf32[slot].astype(jnp.bfloat16)
        qv = q_ref[...]
        w0 = 128 * jnp.maximum(0, 2 * g - 1)

        kw = k_ref[pl.ds(w0, WIN), :]
        k0 = k_ref[0:GW, :]
        s0 = _dot(qv, k0, ((1,), (1,)))
        s1 = _dot(qv, kw, ((1,), (1,)))
        s = jnp.concatenate([s0, s1], axis=1) * SCALE
        s = s + b_ref[...].astype(jnp.float32)
        e = jnp.exp(s).astype(jnp.bfloat16)
        denom = jnp.sum(e.astype(jnp.float32), axis=1, keepdims=True)
        ctx = (_dot(e[:, :GW], v_ref[0:GW, :], ((1,), (0,)))
               + _dot(e[:, GW:], v_ref[pl.ds(w0, WIN), :],
                      ((1,), (0,))))
        o_ref[...] = (ctx / denom).astype(jnp.bfloat16)

        @pl.when(g == 0)
        def _():
            q32 = qv[0:NGLOB, :]
            s32 = _dot(q32, k_ref[...], ((1,), (1,))) * SCALE
            e32 = jnp.exp(s32).astype(jnp.bfloat16)
            d32 = jnp.sum(e32.astype(jnp.float32), axis=1, keepdims=True)
            ctx32 = _dot(e32, v_ref[...], ((1,), (0,)))
            o_ref[0:NGLOB, :] = (ctx32 / d32).astype(jnp.bfloat16)

    return pl.pallas_call(
        body,
        grid=(HQ_PER, N_DEV),
        out_shape=jax.ShapeDtypeStruct((SQ, HD_PER), jnp.bfloat16),
        in_specs=[
            pl.BlockSpec((SQ_PER, DH), lambda h, g: (g, h)),
            pl.BlockSpec(memory_space=pl.ANY),
            pl.BlockSpec(memory_space=pl.ANY),
            pl.BlockSpec((SQ_PER, GW + WIN), lambda h, g: (g, 0)),
        ],
        out_specs=pl.BlockSpec((SQ_PER, DH), lambda h, g: (g, h)),
        scratch_shapes=[
            pltpu.VMEM((2, SKV, DH), jnp.float32),
            pltpu.VMEM((2, SKV, DH), jnp.float32),
            pltpu.VMEM((SKV, DH), jnp.bfloat16),
            pltpu.VMEM((SKV, DH), jnp.bfloat16),
            pltpu.SemaphoreType.DMA((2,)),
            pltpu.SemaphoreType.DMA((2,)),
        ],
    )(Q, K_f32, V_f32, bias)


def _rs_oproj(ctx, Wo_b):

    def body(c_ref, wo_ref, out_ref, cw_s, cw_r, ccw_s, ccw_r,
             cw_ssem, cw_rsem, ccw_ssem, ccw_rsem):
        p = lax.axis_index("i")
        left = lax.rem(p - 1 + N_DEV, N_DEV)
        right = lax.rem(p + 1, N_DEV)
        _neighbor_barrier(left, right)

        wo = wo_ref[...]

        def partial(c):
            c = lax.rem(c + 2 * N_DEV, N_DEV)
            return _dot(c_ref[pl.ds(c * SQ_PER, SQ_PER), :], wo, ((1,), (0,)))

        cw_s[0] = partial(p + 4).astype(jnp.bfloat16)
        ccw_s[0] = partial(p - 3).astype(jnp.bfloat16)
        own_last = None
        for s in range(4):
            cw = pltpu.make_async_remote_copy(
                src_ref=cw_s.at[s], dst_ref=cw_r.at[s],
                send_sem=cw_ssem.at[s], recv_sem=cw_rsem.at[s],
                device_id=(right,), device_id_type=_DeviceIdType.MESH,
            )
            cw.start()
            if s < 3:
                ccw = pltpu.make_async_remote_copy(
                    src_ref=ccw_s.at[s], dst_ref=ccw_r.at[s],
                    send_sem=ccw_ssem.at[s], recv_sem=ccw_rsem.at[s],
                    device_id=(left,), device_id_type=_DeviceIdType.MESH,
                )
                ccw.start()
            nxt_cw = partial(p + 3 - s) if s < 3 else None
            nxt_ccw = partial(p - 2 + s) if s < 2 else None
            if s == 3:
                own_last = partial(p)
            cw.wait()
            if s < 3:
                ccw.wait()
                cw_s[s + 1] = (cw_r[s].astype(jnp.float32)
                               + nxt_cw).astype(jnp.bfloat16)
            if s < 2:
                ccw_s[s + 1] = (ccw_r[s].astype(jnp.float32)
                                + nxt_ccw).astype(jnp.bfloat16)
        out_ref[...] = (cw_r[3].astype(jnp.float32)
                        + ccw_r[2].astype(jnp.float32) + own_last)

    return pl.pallas_call(
        body,
        out_shape=jax.ShapeDtypeStruct((SQ_PER, DM), jnp.float32),
        in_specs=[pl.BlockSpec(memory_space=pltpu.VMEM)] * 2,
        out_specs=pl.BlockSpec(memory_space=pltpu.VMEM),
        scratch_shapes=[
            pltpu.VMEM((4, SQ_PER, DM), jnp.bfloat16),
            pltpu.VMEM((4, SQ_PER, DM), jnp.bfloat16),
            pltpu.VMEM((3, SQ_PER, DM), jnp.bfloat16),
            pltpu.VMEM((3, SQ_PER, DM), jnp.bfloat16),
            pltpu.SemaphoreType.DMA((4,)),
            pltpu.SemaphoreType.DMA((4,)),
            pltpu.SemaphoreType.DMA((3,)),
            pltpu.SemaphoreType.DMA((3,)),
        ],
        compiler_params=_CompilerParams(collective_id=1),
    )(ctx, Wo_b)


def kernel(x, Wq, K_ext, V_ext, Wo):
    x2 = x[0].astype(jnp.bfloat16)
    Wq_b = Wq.astype(jnp.bfloat16)
    Wo_b = Wo.astype(jnp.bfloat16)

    Q = _ag_qproj(x2, Wq_b)
    ctx = _attention(Q, K_ext[0], V_ext[0], _window_bias())
    out = _rs_oproj(ctx, Wo_b)
    return out[None]
